# baseline (speedup 1.0000x reference)
import functools

import jax
import jax.numpy as jnp
from jax.experimental import pallas as pl
from jax.experimental.pallas import tpu as pltpu

_CP = pltpu.CompilerParams(vmem_limit_bytes=67_000_000)

_BM1 = 400
_BM23 = 1008


def _pass1_body(adj_ref, x_ref, w1_ref, b1_ref, w2_ref, g2_ref, adjb_ref,
                xb_ref):
    @pl.when(pl.program_id(0) == 0)
    def _cast_x():
        xb_ref[...] = x_ref[...].astype(jnp.bfloat16)

    ab = adj_ref[...].astype(jnp.bfloat16)
    adjb_ref[...] = ab
    t = jnp.dot(ab, xb_ref[...], preferred_element_type=jnp.float32)
    h = jnp.maximum(
        jnp.dot(t, w1_ref[...], preferred_element_type=jnp.float32) + b1_ref[...],
        0.0,
    )
    g2_ref[...] = jnp.dot(h, w2_ref[...], preferred_element_type=jnp.float32).astype(
        jnp.bfloat16
    )


def _pass23_body(adjb_ref, g2_ref, b2_ref, w3_ref, b3_ref, out_ref, g3_ref,
                 *, n, bm):
    p = pl.program_id(0)
    i = pl.program_id(1)

    @pl.when(p == 0)
    def _layer2():
        t = jnp.dot(adjb_ref[...], g2_ref[...],
                    preferred_element_type=jnp.float32)
        h = jnp.maximum(t + b2_ref[...], 0.0)
        g3_ref[pl.ds(i * bm, bm), :] = jnp.dot(
            h, w3_ref[...], preferred_element_type=jnp.float32
        ).astype(jnp.bfloat16)

    @pl.when(p == 1)
    def _layer3():
        out_ref[...] = (
            jnp.dot(adjb_ref[...], g3_ref[:n, :],
                    preferred_element_type=jnp.float32)
            + b3_ref[...]
        )


@jax.jit
def kernel(x, adj, labels, W1, b1, W2, b2, W3, b3):
    del labels
    n, nfeat = x.shape
    nhid = W1.shape[1]
    ncls = W3.shape[1]
    bm1 = min(_BM1, n)
    bm23 = min(_BM23, n)
    nblk = pl.cdiv(n, bm23)

    b1r = b1.reshape(1, nhid)
    b2r = b2.reshape(1, nhid)
    b3r = b3.reshape(1, ncls)

    full1 = lambda shape: pl.BlockSpec(shape, lambda i: (0, 0))
    g2, adjb = pl.pallas_call(
        _pass1_body,
        grid=(pl.cdiv(n, bm1),),
        compiler_params=_CP,
        in_specs=[
            pl.BlockSpec((bm1, n), lambda i: (i, 0)),
            full1((n, nfeat)),
            full1((nfeat, nhid)),
            full1((1, nhid)),
            full1((nhid, nhid)),
        ],
        out_specs=[
            pl.BlockSpec((bm1, nhid), lambda i: (i, 0)),
            pl.BlockSpec((bm1, n), lambda i: (i, 0)),
        ],
        out_shape=[
            jax.ShapeDtypeStruct((n, nhid), jnp.bfloat16),
            jax.ShapeDtypeStruct((n, n), jnp.bfloat16),
        ],
        scratch_shapes=[pltpu.VMEM((n, nfeat), jnp.bfloat16)],
    )(adj, x, W1, b1r, W2)

    full2 = lambda shape: pl.BlockSpec(shape, lambda p, i: (0, 0))
    out = pl.pallas_call(
        functools.partial(_pass23_body, n=n, bm=bm23),
        grid=(2, nblk),
        compiler_params=_CP,
        in_specs=[
            pl.BlockSpec((bm23, n), lambda p, i: (i, 0)),
            full2((n, nhid)),
            full2((1, nhid)),
            full2((nhid, ncls)),
            full2((1, ncls)),
        ],
        out_specs=pl.BlockSpec((bm23, ncls), lambda p, i: (p * i, 0)),
        out_shape=jax.ShapeDtypeStruct((n, ncls), jnp.float32),
        scratch_shapes=[pltpu.VMEM((nblk * bm23, ncls), jnp.bfloat16)],
    )(adjb, g2, b2r, W3, b3r)
    return out


# speedup vs baseline: 1.0183x; 1.0183x over previous
import functools

import jax
import jax.numpy as jnp
from jax.experimental import pallas as pl
from jax.experimental.pallas import tpu as pltpu

_CP = pltpu.CompilerParams(vmem_limit_bytes=67_000_000)

_BM1 = 400
_BM23 = 1120


def _pass1_body(adj_ref, x_ref, w1_ref, b1_ref, w2_ref, g2_ref, adjb_ref,
                xb_ref):
    @pl.when(pl.program_id(0) == 0)
    def _cast_x():
        xb_ref[...] = x_ref[...].astype(jnp.bfloat16)

    ab = adj_ref[...].astype(jnp.bfloat16)
    adjb_ref[...] = ab
    t = jnp.dot(ab, xb_ref[...], preferred_element_type=jnp.float32)
    h = jnp.maximum(
        jnp.dot(t, w1_ref[...], preferred_element_type=jnp.float32) + b1_ref[...],
        0.0,
    )
    g2_ref[...] = jnp.dot(h, w2_ref[...], preferred_element_type=jnp.float32).astype(
        jnp.bfloat16
    )


def _pass23_body(adjb_ref, g2_ref, b2_ref, w3_ref, b3_ref, out_ref, g3_ref,
                 *, n, bm):
    p = pl.program_id(0)
    i = pl.program_id(1)

    @pl.when(p == 0)
    def _layer2():
        t = jnp.dot(adjb_ref[...], g2_ref[...],
                    preferred_element_type=jnp.float32)
        h = jnp.maximum(t + b2_ref[...], 0.0)
        g3_ref[pl.ds(i * bm, bm), :] = jnp.dot(
            h, w3_ref[...], preferred_element_type=jnp.float32
        ).astype(jnp.bfloat16)

    @pl.when(p == 1)
    def _layer3():
        out_ref[...] = (
            jnp.dot(adjb_ref[...], g3_ref[:n, :],
                    preferred_element_type=jnp.float32)
            + b3_ref[...]
        )


@jax.jit
def kernel(x, adj, labels, W1, b1, W2, b2, W3, b3):
    del labels
    n, nfeat = x.shape
    nhid = W1.shape[1]
    ncls = W3.shape[1]
    bm1 = min(_BM1, n)
    bm23 = min(_BM23, n)
    nblk = pl.cdiv(n, bm23)

    b1r = b1.reshape(1, nhid)
    b2r = b2.reshape(1, nhid)
    b3r = b3.reshape(1, ncls)

    full1 = lambda shape: pl.BlockSpec(shape, lambda i: (0, 0))
    g2, adjb = pl.pallas_call(
        _pass1_body,
        grid=(pl.cdiv(n, bm1),),
        compiler_params=_CP,
        in_specs=[
            pl.BlockSpec((bm1, n), lambda i: (i, 0)),
            full1((n, nfeat)),
            full1((nfeat, nhid)),
            full1((1, nhid)),
            full1((nhid, nhid)),
        ],
        out_specs=[
            pl.BlockSpec((bm1, nhid), lambda i: (i, 0)),
            pl.BlockSpec((bm1, n), lambda i: (i, 0)),
        ],
        out_shape=[
            jax.ShapeDtypeStruct((n, nhid), jnp.bfloat16),
            jax.ShapeDtypeStruct((n, n), jnp.bfloat16),
        ],
        scratch_shapes=[pltpu.VMEM((n, nfeat), jnp.bfloat16)],
    )(adj, x, W1, b1r, W2)

    full2 = lambda shape: pl.BlockSpec(shape, lambda p, i: (0, 0))
    out = pl.pallas_call(
        functools.partial(_pass23_body, n=n, bm=bm23),
        grid=(2, nblk),
        compiler_params=_CP,
        in_specs=[
            pl.BlockSpec((bm23, n), lambda p, i: (i, 0)),
            full2((n, nhid)),
            full2((1, nhid)),
            full2((nhid, ncls)),
            full2((1, ncls)),
        ],
        out_specs=pl.BlockSpec((bm23, ncls), lambda p, i: (p * i, 0)),
        out_shape=jax.ShapeDtypeStruct((n, ncls), jnp.float32),
        scratch_shapes=[pltpu.VMEM((nblk * bm23, ncls), jnp.bfloat16)],
    )(adjb, g2, b2r, W3, b3r)
    return out
